# Initial kernel scaffold; baseline (speedup 1.0000x reference)
import jax
import jax.numpy as jnp
from jax.experimental import pallas as pl


def kernel(x, edge_index, W1, a_src1, a_dst1, b1, W2, a_src2, a_dst2, b2):
    raise NotImplementedError("write your pallas kernel here")


# speedup vs baseline: 71.0210x; 71.0210x over previous
import functools

import jax
import jax.numpy as jnp
from jax import lax
from jax.experimental import pallas as pl
from jax.experimental.pallas import tpu as pltpu
from jax.experimental.pallas import tpu_sc as plsc

_N = 10000
_E = 320000
_IN = 128
_HID = 16
_HEADS = 8
_OUT = 64

_NC = 2
_NS = 16
_NW = _NC * _NS
_K = 200
_EPW = _E // _NW
_NPAD = 10240
_ROWS_PT = _NPAD // _NS
_ZSTEP = 160

_MROWS = 2000

_HI = lax.Precision.HIGHEST
_DN = (((1,), (0,)), ((), ()))


def _tc1_body(x_ref, w_ref, ms_ref, md_ref, h_ref, as_ref, ad_ref):
    h = lax.dot_general(x_ref[...], w_ref[...], _DN, precision=_HI)
    h_ref[...] = h
    as_ref[...] = lax.dot_general(h, ms_ref[...], _DN, precision=_HI)
    ad_ref[...] = lax.dot_general(h, md_ref[...], _DN, precision=_HI)


def _tc2_body(acc_ref, den_ref, e1_ref, b1_ref, w2_ref, m2s_ref, m2d_ref,
              h2_ref, as2_ref, ad2_ref):
    accsum = acc_ref[0] + acc_ref[1]
    densum = den_ref[0] + den_ref[1]
    den_exp = lax.dot_general(densum, e1_ref[...], _DN, precision=_HI)
    h = accsum / (den_exp + 1e-16) + b1_ref[...]
    h = jnp.where(h >= 0.0, h, 0.2 * h)
    h2 = lax.dot_general(h, w2_ref[...], _DN, precision=_HI)
    h2_ref[...] = h2
    as2_ref[...] = lax.dot_general(h2, m2s_ref[...], _DN, precision=_HI)
    ad2_ref[...] = lax.dot_general(h2, m2d_ref[...], _DN, precision=_HI)


def _tc3_body(acc_ref, den_ref, e2_ref, b2_ref, o_ref):
    accsum = acc_ref[0] + acc_ref[1]
    densum = den_ref[0] + den_ref[1]
    den_exp = lax.dot_general(densum, e2_ref[...], _DN, precision=_HI)
    o_ref[...] = accsum / (den_exp + 1e-16) + b2_ref[...]


def _make_edge_kernel(C, H):
    nv = C // 16
    grp = (C // H) // 16
    mesh = plsc.VectorSubcoreMesh(core_axis_name="c", subcore_axis_name="s",
                                  num_cores=_NC, num_subcores=_NS)

    @functools.partial(
        pl.kernel,
        out_type=[jax.ShapeDtypeStruct((_NC, _NPAD, C), jnp.float32),
                  jax.ShapeDtypeStruct((_NC, _NPAD, 16), jnp.float32)],
        mesh=mesh,
        compiler_params=pltpu.CompilerParams(use_tc_tiling_on_sc=False),
        scratch_types=[
            pltpu.VMEM((_K,), jnp.int32),
            pltpu.VMEM((_K,), jnp.int32),
            pltpu.VMEM((_K, 16), jnp.float32),
            pltpu.VMEM((_K, 16), jnp.float32),
            pltpu.VMEM((_K, 16), jnp.float32),
            pltpu.VMEM((_K, C), jnp.float32),
            pltpu.VMEM_SHARED((_NPAD, C), jnp.float32),
            pltpu.VMEM_SHARED((_NPAD, 16), jnp.float32),
            pltpu.SemaphoreType.DMA,
            pltpu.SemaphoreType.DMA,
            pltpu.SemaphoreType.DMA,
        ],
    )
    def edge_kernel(h_hbm, asp_hbm, adp_hbm, src_hbm, dst_hbm,
                    acc_hbm, den_hbm,
                    idx_s, idx_d, gs, gd, pbuf, hs, acc_sh, den_sh,
                    sem_a, sem_b, sem_h):
        c = lax.axis_index("c")
        s = lax.axis_index("s")
        wid = s * _NC + c

        zv = jnp.zeros((16,), jnp.float32)

        @pl.loop(0, _K)
        def _(k):
            pbuf[k, :] = zv
            for j in range(nv):
                hs[k, pl.ds(j * 16, 16)] = zv

        base = s * _ROWS_PT

        @pl.loop(0, _ROWS_PT, step=_ZSTEP)
        def _(r):
            pltpu.sync_copy(hs.at[pl.ds(0, _ZSTEP)],
                            acc_sh.at[pl.ds(base + r, _ZSTEP)])
            pltpu.sync_copy(pbuf.at[pl.ds(0, _ZSTEP)],
                            den_sh.at[pl.ds(base + r, _ZSTEP)])

        plsc.subcore_barrier()

        @pl.loop(0, _EPW, step=_K)
        def _(i):
            e0 = wid * _EPW + i
            pltpu.sync_copy(src_hbm.at[pl.ds(e0, _K)], idx_s)
            pltpu.sync_copy(dst_hbm.at[pl.ds(e0, _K)], idx_d)
            cph = pltpu.async_copy(h_hbm.at[idx_s], hs, sem_h)
            cpa = pltpu.async_copy(asp_hbm.at[idx_s], gs, sem_a)
            cpb = pltpu.async_copy(adp_hbm.at[idx_d], gd, sem_b)
            cpa.wait()
            cpb.wait()

            @pl.loop(0, _K)
            def _(k):
                e = gs[k, :] + gd[k, :]
                e = jnp.where(e >= 0.0, e, 0.2 * e)
                pbuf[k, :] = jnp.exp(e)

            pltpu.sync_copy(pbuf, den_sh.at[idx_d], add=True)
            cph.wait()

            @pl.loop(0, _K)
            def _(k):
                pv = pbuf[k, :]
                for j in range(nv):
                    pk = pv[j // grp]
                    hs[k, pl.ds(j * 16, 16)] = hs[k, pl.ds(j * 16, 16)] * pk

            pltpu.sync_copy(hs, acc_sh.at[idx_d], add=True)

        plsc.subcore_barrier()

        pltpu.sync_copy(acc_sh.at[pl.ds(base, _ROWS_PT)],
                        acc_hbm.at[c, pl.ds(base, _ROWS_PT)])
        pltpu.sync_copy(den_sh.at[pl.ds(base, _ROWS_PT)],
                        den_hbm.at[c, pl.ds(base, _ROWS_PT)])

    return edge_kernel


_edge_cache = {}


def _edge_kernel(C, H):
    if (C, H) not in _edge_cache:
        _edge_cache[(C, H)] = _make_edge_kernel(C, H)
    return _edge_cache[(C, H)]


def kernel(x, edge_index, W1, a_src1, a_dst1, b1, W2, a_src2, a_dst2, b2):
    src = edge_index[0]
    dst = edge_index[1]
    f32 = jnp.float32

    a1s = a_src1.reshape(_HEADS, _HID).astype(f32)
    a1d = a_dst1.reshape(_HEADS, _HID).astype(f32)
    eye8 = jnp.eye(_HEADS, dtype=f32)
    m1s = jnp.pad((a1s[:, :, None] * eye8[:, None, :]).reshape(_IN, _HEADS),
                  ((0, 0), (0, 16 - _HEADS)))
    m1d = jnp.pad((a1d[:, :, None] * eye8[:, None, :]).reshape(_IN, _HEADS),
                  ((0, 0), (0, 16 - _HEADS)))
    e1 = jnp.pad(jnp.kron(eye8, jnp.ones((1, _HID), f32)),
                 ((0, 16 - _HEADS), (0, 0)))
    e2 = jnp.pad(jnp.ones((1, _OUT), f32), ((0, 15), (0, 0)))
    m2s = jnp.pad(a_src2.reshape(_OUT, 1).astype(f32), ((0, 0), (0, 15)))
    m2d = jnp.pad(a_dst2.reshape(_OUT, 1).astype(f32), ((0, 0), (0, 15)))

    grid = _N // _MROWS
    c1 = _HEADS * _HID

    h1, as1, ad1 = pl.pallas_call(
        _tc1_body,
        grid=(grid,),
        in_specs=[
            pl.BlockSpec((_MROWS, _IN), lambda i: (i, 0)),
            pl.BlockSpec((_IN, c1), lambda i: (0, 0)),
            pl.BlockSpec((c1, 16), lambda i: (0, 0)),
            pl.BlockSpec((c1, 16), lambda i: (0, 0)),
        ],
        out_specs=[
            pl.BlockSpec((_MROWS, c1), lambda i: (i, 0)),
            pl.BlockSpec((_MROWS, 16), lambda i: (i, 0)),
            pl.BlockSpec((_MROWS, 16), lambda i: (i, 0)),
        ],
        out_shape=[
            jax.ShapeDtypeStruct((_N, c1), f32),
            jax.ShapeDtypeStruct((_N, 16), f32),
            jax.ShapeDtypeStruct((_N, 16), f32),
        ],
    )(x, W1, m1s, m1d)

    acc1, den1 = _edge_kernel(c1, _HEADS)(h1, as1, ad1, src, dst)

    h2, as2, ad2 = pl.pallas_call(
        _tc2_body,
        grid=(grid,),
        in_specs=[
            pl.BlockSpec((_NC, _MROWS, c1), lambda i: (0, i, 0)),
            pl.BlockSpec((_NC, _MROWS, 16), lambda i: (0, i, 0)),
            pl.BlockSpec((16, c1), lambda i: (0, 0)),
            pl.BlockSpec((1, c1), lambda i: (0, 0)),
            pl.BlockSpec((c1, _OUT), lambda i: (0, 0)),
            pl.BlockSpec((_OUT, 16), lambda i: (0, 0)),
            pl.BlockSpec((_OUT, 16), lambda i: (0, 0)),
        ],
        out_specs=[
            pl.BlockSpec((_MROWS, _OUT), lambda i: (i, 0)),
            pl.BlockSpec((_MROWS, 16), lambda i: (i, 0)),
            pl.BlockSpec((_MROWS, 16), lambda i: (i, 0)),
        ],
        out_shape=[
            jax.ShapeDtypeStruct((_N, _OUT), f32),
            jax.ShapeDtypeStruct((_N, 16), f32),
            jax.ShapeDtypeStruct((_N, 16), f32),
        ],
    )(acc1, den1, e1, b1.reshape(1, c1), W2, m2s, m2d)

    acc2, den2 = _edge_kernel(_OUT, 1)(h2, as2, ad2, src, dst)

    out = pl.pallas_call(
        _tc3_body,
        grid=(grid,),
        in_specs=[
            pl.BlockSpec((_NC, _MROWS, _OUT), lambda i: (0, i, 0)),
            pl.BlockSpec((_NC, _MROWS, 16), lambda i: (0, i, 0)),
            pl.BlockSpec((16, _OUT), lambda i: (0, 0)),
            pl.BlockSpec((1, _OUT), lambda i: (0, 0)),
        ],
        out_specs=pl.BlockSpec((_MROWS, _OUT), lambda i: (i, 0)),
        out_shape=jax.ShapeDtypeStruct((_N, _OUT), f32),
    )(acc2, den2, e2, b2.reshape(1, _OUT))

    return out
